# baseline (speedup 1.0000x reference)
import functools

import jax
import jax.numpy as jnp
from jax.experimental import pallas as pl

N = 10000
E = 320000
F_IN = 128
A = 10000
K = 3

_KB = 2048
_AB = 1280
_K_BLOCKS = 5
_A_BLOCKS = 8
_KTOT = 10003


def _head_body(concat_ref, aW_ref, ab_ref, cW_ref, cb_ref, logits_ref, values_ref):
    a = pl.program_id(0)
    k = pl.program_id(1)
    c = concat_ref[...]
    w = aW_ref[...]
    rows_valid = (jax.lax.broadcasted_iota(jnp.int32, (_KB, 1), 0) + k * _KB) < _KTOT
    w = jnp.where(rows_valid, w, 0.0)
    part = jax.lax.dot_general(c, w, (((1,), (0,)), ((), ())),
                               preferred_element_type=jnp.float32)

    @pl.when(k == 0)
    def _():
        logits_ref[...] = ab_ref[...]

    logits_ref[...] += part

    @pl.when(a == 0)
    def _():
        cw = cW_ref[...]
        cw = jnp.where(rows_valid, cw, 0.0)
        cpart = jax.lax.dot_general(c, cw, (((1,), (0,)), ((), ())),
                                    preferred_element_type=jnp.float32)

        @pl.when(k == 0)
        def _():
            values_ref[...] = cb_ref[...]

        values_ref[...] += cpart


def _head(concat_pad, actor_W, actor_b, critic_W, critic_b):
    grid = (_A_BLOCKS, _K_BLOCKS)
    return pl.pallas_call(
        _head_body,
        grid=grid,
        in_specs=[
            pl.BlockSpec((1, _KB), lambda a, k: (0, k)),
            pl.BlockSpec((_KB, _AB), lambda a, k: (k, a)),
            pl.BlockSpec((1, _AB), lambda a, k: (0, a)),
            pl.BlockSpec((_KB, 1), lambda a, k: (k, 0)),
            pl.BlockSpec((1, 1), lambda a, k: (0, 0)),
        ],
        out_specs=[
            pl.BlockSpec((1, _AB), lambda a, k: (0, a)),
            pl.BlockSpec((1, 1), lambda a, k: (0, 0)),
        ],
        out_shape=[
            jax.ShapeDtypeStruct((1, A), jnp.float32),
            jax.ShapeDtypeStruct((1, 1), jnp.float32),
        ],
    )(concat_pad, actor_W, actor_b, critic_W, critic_b)


def _cheb(x, edge_index, W, b):
    row = edge_index[0]
    col = edge_index[1]
    deg = jax.ops.segment_sum(jnp.ones((E,), x.dtype), row, num_segments=N)
    dis = jnp.where(deg > 0, 1.0 / jnp.sqrt(jnp.maximum(deg, 1e-12)), 0.0)
    norm = -dis[row] * dis[col]

    def prop(t):
        return jax.ops.segment_sum(norm[:, None] * t[col], row, num_segments=N)

    Tx0 = x
    out = Tx0 @ W[0]
    Tx1 = prop(Tx0)
    out = out + Tx1 @ W[1]
    Tx2 = 2.0 * prop(Tx1) - Tx0
    out = out + Tx2 @ W[2]
    return out + b


def kernel(substrate_features, edge_index, v_CPU_request, v_BW_demand,
           pending_v_nodes, W1, b1, W2, b2, W3, b3,
           actor_W, actor_b, critic_W, critic_b):
    x = substrate_features[0]
    h = jnp.tanh(_cheb(x, edge_index, W1, b1))
    h = jnp.tanh(_cheb(h, edge_index, W2, b2))
    h = jnp.tanh(_cheb(h, edge_index, W3, b3))
    gcn = h[:, 0]
    concat = jnp.concatenate([
        gcn, v_CPU_request, v_BW_demand, pending_v_nodes,
        jnp.zeros((_K_BLOCKS * _KB - _KTOT,), jnp.float32),
    ])[None, :]
    logits, values = _head(concat, actor_W, actor_b[None, :],
                           critic_W, critic_b[None, :])
    return logits, values


# speedup vs baseline: 5.1321x; 5.1321x over previous
import functools

import jax
import jax.numpy as jnp
from jax import lax
from jax.experimental import pallas as pl
from jax.experimental.pallas import tpu as pltpu
from jax.experimental.pallas import tpu_sc as plsc

N = 10000
E = 320000
A = 10000

NC, NS = 2, 16
NW = NC * NS
CHUNK = 128
EPW = 10240
NCHUNK = EPW // CHUNK
E_PAD = NW * EPW
N_PAD = 10240
ROWS_PT = N_PAD // NS

_KB = 2048
_AB = 1280
_K_BLOCKS = 5
_A_BLOCKS = 8
_KTOT = N + 3


def _make_prop(F):
    mesh = plsc.VectorSubcoreMesh(core_axis_name="c", subcore_axis_name="s")

    @functools.partial(
        pl.kernel,
        out_type=jax.ShapeDtypeStruct((NC, N_PAD, F), jnp.float32),
        mesh=mesh,
        scratch_types=[
            pltpu.VMEM((NCHUNK, CHUNK), jnp.int32),
            pltpu.VMEM((NCHUNK, CHUNK), jnp.int32),
            pltpu.VMEM((CHUNK, F), jnp.float32),
            pltpu.VMEM_SHARED((N_PAD, F), jnp.float32),
        ],
        compiler_params=pltpu.CompilerParams(use_tc_tiling_on_sc=False),
    )
    def prop(col_hbm, row_hbm, u_hbm, zeros_hbm, out_hbm,
             col_v, row_v, rows_v, acc):
        cid = lax.axis_index("c")
        sid = lax.axis_index("s")
        w = cid * NS + sid
        pltpu.sync_copy(zeros_hbm, acc.at[pl.ds(sid * ROWS_PT, ROWS_PT)])
        pltpu.sync_copy(col_hbm.at[w], col_v)
        pltpu.sync_copy(row_hbm.at[w], row_v)
        plsc.subcore_barrier()

        def body(c, carry):
            pltpu.sync_copy(u_hbm.at[col_v.at[c]], rows_v)
            pltpu.sync_copy(rows_v, acc.at[row_v.at[c]], add=True)
            return carry

        lax.fori_loop(0, NCHUNK, body, 0)
        plsc.subcore_barrier()
        pltpu.sync_copy(acc.at[pl.ds(sid * ROWS_PT, ROWS_PT)],
                        out_hbm.at[cid, pl.ds(sid * ROWS_PT, ROWS_PT)])

    return prop


_DEG_F = 8


def _make_deg():
    mesh = plsc.VectorSubcoreMesh(core_axis_name="c", subcore_axis_name="s")

    @functools.partial(
        pl.kernel,
        out_type=jax.ShapeDtypeStruct((NC, N_PAD, _DEG_F), jnp.float32),
        mesh=mesh,
        scratch_types=[
            pltpu.VMEM((NCHUNK, CHUNK), jnp.int32),
            pltpu.VMEM((CHUNK, _DEG_F), jnp.float32),
            pltpu.VMEM_SHARED((N_PAD, _DEG_F), jnp.float32),
        ],
        compiler_params=pltpu.CompilerParams(use_tc_tiling_on_sc=False),
    )
    def deg(row_hbm, ones_hbm, zeros_hbm, out_hbm, row_v, ones_v, acc):
        cid = lax.axis_index("c")
        sid = lax.axis_index("s")
        w = cid * NS + sid
        pltpu.sync_copy(zeros_hbm, acc.at[pl.ds(sid * ROWS_PT, ROWS_PT)])
        pltpu.sync_copy(row_hbm.at[w], row_v)
        pltpu.sync_copy(ones_hbm, ones_v)
        plsc.subcore_barrier()

        def body(c, carry):
            pltpu.sync_copy(ones_v, acc.at[row_v.at[c]], add=True)
            return carry

        lax.fori_loop(0, NCHUNK, body, 0)
        plsc.subcore_barrier()
        pltpu.sync_copy(acc.at[pl.ds(sid * ROWS_PT, ROWS_PT)],
                        out_hbm.at[cid, pl.ds(sid * ROWS_PT, ROWS_PT)])

    return deg


_prop128 = _make_prop(128)
_prop64 = _make_prop(64)
_prop32 = _make_prop(32)
_deg_kernel = _make_deg()


def _dot(a, b):
    return jax.lax.dot_general(a, b, (((1,), (0,)), ((), ())),
                               preferred_element_type=jnp.float32)


def _stage_a_body(degp_ref, x_ref, w_ref, dis_ref, u0_ref, acc0_ref):
    d = degp_ref[0] + degp_ref[1]
    deg = d[:N, 0:1]
    dis = jnp.where(deg > 0, 1.0 / jnp.sqrt(jnp.maximum(deg, 1e-12)), 0.0)
    x = x_ref[...]
    dis_ref[...] = dis
    u0_ref[...] = x * dis
    acc0_ref[...] = _dot(x, w_ref[...])


def _stage_a(degp, x, w10):
    return pl.pallas_call(
        _stage_a_body,
        out_shape=[
            jax.ShapeDtypeStruct((N, 1), jnp.float32),
            jax.ShapeDtypeStruct((N, 128), jnp.float32),
            jax.ShapeDtypeStruct((N, w10.shape[1]), jnp.float32),
        ],
    )(degp, x, w10)


def _p1_body(fin, fpad, sp_ref, dis_ref, accp_ref, w_ref, acc_ref, u_ref):
    s = sp_ref[0] + sp_ref[1]
    dis = dis_ref[...]
    t = -dis * s[:N, :fin]
    acc_ref[...] = accp_ref[...] + _dot(t, w_ref[...])
    u = dis * t
    if fpad > fin:
        u = jnp.concatenate([u, jnp.zeros((N, fpad - fin), jnp.float32)], axis=1)
    u_ref[...] = u


def _stage_p1(sp, dis, accp, w, fin, fpad):
    return pl.pallas_call(
        functools.partial(_p1_body, fin, fpad),
        out_shape=[
            jax.ShapeDtypeStruct((N, w.shape[1]), jnp.float32),
            jax.ShapeDtypeStruct((N, fpad), jnp.float32),
        ],
    )(sp, dis, accp, w)


def _p2_body(fin, fpad_next, sp_ref, dis_ref, accp_ref, w_ref, b_ref,
             tx0_ref, wn_ref, h_ref, u_ref, accn_ref):
    s = sp_ref[0] + sp_ref[1]
    dis = dis_ref[...]
    p = -dis * s[:N, :fin]
    tx2 = 2.0 * p - tx0_ref[...]
    h = jnp.tanh(accp_ref[...] + _dot(tx2, w_ref[...]) + b_ref[...])
    h_ref[...] = h
    u = dis * h
    fo = h.shape[1]
    if fpad_next > fo:
        u = jnp.concatenate([u, jnp.zeros((N, fpad_next - fo), jnp.float32)],
                            axis=1)
    u_ref[...] = u
    accn_ref[...] = _dot(h, wn_ref[...])


def _stage_p2(sp, dis, accp, w, b, tx0, wn, fin, fpad_next):
    fo = w.shape[1]
    return pl.pallas_call(
        functools.partial(_p2_body, fin, fpad_next),
        out_shape=[
            jax.ShapeDtypeStruct((N, fo), jnp.float32),
            jax.ShapeDtypeStruct((N, fpad_next), jnp.float32),
            jax.ShapeDtypeStruct((N, wn.shape[1]), jnp.float32),
        ],
    )(sp, dis, accp, w, b, tx0, wn)


def _p2_last_body(fin, sp_ref, dis_ref, accp_ref, w_ref, b_ref, tx0_ref,
                  h_ref):
    s = sp_ref[0] + sp_ref[1]
    dis = dis_ref[...]
    p = -dis * s[:N, :fin]
    tx2 = 2.0 * p - tx0_ref[...]
    h_ref[...] = jnp.tanh(accp_ref[...] + _dot(tx2, w_ref[...]) + b_ref[...])


def _stage_p2_last(sp, dis, accp, w, b, tx0, fin):
    return pl.pallas_call(
        functools.partial(_p2_last_body, fin),
        out_shape=jax.ShapeDtypeStruct((N, w.shape[1]), jnp.float32),
    )(sp, dis, accp, w, b, tx0)


def _head_body(concat_ref, aW_ref, ab_ref, cW_ref, cb_ref, logits_ref,
               values_ref):
    a = pl.program_id(0)
    k = pl.program_id(1)
    c = concat_ref[...]
    w = aW_ref[...]
    rows_valid = (jax.lax.broadcasted_iota(jnp.int32, (_KB, 1), 0)
                  + k * _KB) < _KTOT
    w = jnp.where(rows_valid, w, 0.0)
    part = _dot(c, w)

    @pl.when(k == 0)
    def _():
        logits_ref[...] = ab_ref[...]

    logits_ref[...] += part

    @pl.when(a == 0)
    def _():
        cw = jnp.where(rows_valid, cW_ref[...], 0.0)
        cpart = _dot(c, cw)

        @pl.when(k == 0)
        def _():
            values_ref[...] = cb_ref[...]

        values_ref[...] += cpart


def _head(concat_pad, actor_W, actor_b, critic_W, critic_b):
    return pl.pallas_call(
        _head_body,
        grid=(_A_BLOCKS, _K_BLOCKS),
        in_specs=[
            pl.BlockSpec((1, _KB), lambda a, k: (0, k)),
            pl.BlockSpec((_KB, _AB), lambda a, k: (k, a)),
            pl.BlockSpec((1, _AB), lambda a, k: (0, a)),
            pl.BlockSpec((_KB, 1), lambda a, k: (k, 0)),
            pl.BlockSpec((1, 1), lambda a, k: (0, 0)),
        ],
        out_specs=[
            pl.BlockSpec((1, _AB), lambda a, k: (0, a)),
            pl.BlockSpec((1, 1), lambda a, k: (0, 0)),
        ],
        out_shape=[
            jax.ShapeDtypeStruct((1, A), jnp.float32),
            jax.ShapeDtypeStruct((1, 1), jnp.float32),
        ],
    )(concat_pad, actor_W, actor_b, critic_W, critic_b)


def kernel(substrate_features, edge_index, v_CPU_request, v_BW_demand,
           pending_v_nodes, W1, b1, W2, b2, W3, b3,
           actor_W, actor_b, critic_W, critic_b):
    x = substrate_features[0]
    row = edge_index[0]
    col = edge_index[1]

    pad = E_PAD - E
    col_p = jnp.concatenate([col, jnp.zeros((pad,), jnp.int32)])
    col_p = col_p.reshape(NW, NCHUNK, CHUNK)
    row_p = jnp.concatenate([row, jnp.full((pad,), N, jnp.int32)])
    row_p = row_p.reshape(NW, NCHUNK, CHUNK)

    z8 = jnp.zeros((ROWS_PT, _DEG_F), jnp.float32)
    z128 = jnp.zeros((ROWS_PT, 128), jnp.float32)
    z64 = jnp.zeros((ROWS_PT, 64), jnp.float32)
    z32 = jnp.zeros((ROWS_PT, 32), jnp.float32)
    ones8 = jnp.concatenate(
        [jnp.ones((CHUNK, 1), jnp.float32),
         jnp.zeros((CHUNK, _DEG_F - 1), jnp.float32)], axis=1)

    degp = _deg_kernel(row_p, ones8, z8)

    dis, u0, acc0 = _stage_a(degp, x, W1[0])
    s1 = _prop128(col_p, row_p, u0, z128)
    acc1, u1 = _stage_p1(s1, dis, acc0, W1[1], 128, 128)
    s2 = _prop128(col_p, row_p, u1, z128)
    h1, u2, acc2 = _stage_p2(s2, dis, acc1, W1[2], b1[None, :], x, W2[0],
                             128, 64)

    s3 = _prop64(col_p, row_p, u2, z64)
    acc3, u3 = _stage_p1(s3, dis, acc2, W2[1], 60, 64)
    s4 = _prop64(col_p, row_p, u3, z64)
    h2, u4, acc4 = _stage_p2(s4, dis, acc3, W2[2], b2[None, :], h1, W3[0],
                             60, 32)

    s5 = _prop32(col_p, row_p, u4, z32)
    acc5, u5 = _stage_p1(s5, dis, acc4, W3[1], 30, 32)
    s6 = _prop32(col_p, row_p, u5, z32)
    h3 = _stage_p2_last(s6, dis, acc5, W3[2], b3[None, :], h2, 30)

    concat = jnp.concatenate([
        h3[:, 0], v_CPU_request, v_BW_demand, pending_v_nodes,
        jnp.zeros((_K_BLOCKS * _KB - _KTOT,), jnp.float32),
    ])[None, :]
    logits, values = _head(concat, actor_W, actor_b[None, :],
                           critic_W, critic_b[None, :])
    return logits, values


# speedup vs baseline: 5.6506x; 1.1010x over previous
import functools

import jax
import jax.numpy as jnp
from jax import lax
from jax.experimental import pallas as pl
from jax.experimental.pallas import tpu as pltpu
from jax.experimental.pallas import tpu_sc as plsc

N = 10000
E = 320000
A = 10000

NC, NS = 2, 16
NW = NC * NS
EPG = 1024
EPW = 10240
NGRP = EPW // EPG
E_PAD = NW * EPW
N_PAD = 10240
ROWS_PT = N_PAD // NS

_KB = 2048
_AB = 1280
_K_BLOCKS = 5
_A_BLOCKS = 8
_KTOT = N + 3


def _make_prop(F, staged):
    mesh = plsc.VectorSubcoreMesh(core_axis_name="c", subcore_axis_name="s")

    scratch = [
        pltpu.VMEM((NGRP, EPG), jnp.int32),
        pltpu.VMEM((NGRP, EPG), jnp.int32),
        pltpu.VMEM((EPG, F), jnp.float32),
        pltpu.VMEM_SHARED((N_PAD, F), jnp.float32),
    ]
    if staged:
        scratch.append(pltpu.VMEM_SHARED((N_PAD, F), jnp.float32))

    @functools.partial(
        pl.kernel,
        out_type=jax.ShapeDtypeStruct((NC, N_PAD, F), jnp.float32),
        mesh=mesh,
        scratch_types=scratch,
        compiler_params=pltpu.CompilerParams(use_tc_tiling_on_sc=False),
    )
    def prop(col_hbm, row_hbm, u_hbm, zeros_hbm, out_hbm,
             col_v, row_v, buf, acc, *maybe_usp):
        cid = lax.axis_index("c")
        sid = lax.axis_index("s")
        w = cid * NS + sid
        sl = pl.ds(sid * ROWS_PT, ROWS_PT)
        pltpu.sync_copy(zeros_hbm, acc.at[sl])
        if staged:
            u_src = maybe_usp[0]
            pltpu.sync_copy(u_hbm.at[sl], u_src.at[sl])
        else:
            u_src = u_hbm
        pltpu.sync_copy(col_hbm.at[w], col_v)
        pltpu.sync_copy(row_hbm.at[w], row_v)
        plsc.subcore_barrier()

        def body(g, carry):
            pltpu.sync_copy(u_src.at[col_v.at[g]], buf)
            pltpu.sync_copy(buf, acc.at[row_v.at[g]], add=True)
            return carry

        lax.fori_loop(0, NGRP, body, 0)
        plsc.subcore_barrier()
        pltpu.sync_copy(acc.at[sl], out_hbm.at[cid, sl])

    return prop


_DEG_F = 8


def _make_deg():
    mesh = plsc.VectorSubcoreMesh(core_axis_name="c", subcore_axis_name="s")

    @functools.partial(
        pl.kernel,
        out_type=jax.ShapeDtypeStruct((NC, N_PAD, _DEG_F), jnp.float32),
        mesh=mesh,
        scratch_types=[
            pltpu.VMEM((NGRP, EPG), jnp.int32),
            pltpu.VMEM((EPG, _DEG_F), jnp.float32),
            pltpu.VMEM_SHARED((N_PAD, _DEG_F), jnp.float32),
        ],
        compiler_params=pltpu.CompilerParams(use_tc_tiling_on_sc=False),
    )
    def deg(row_hbm, ones_hbm, zeros_hbm, out_hbm, row_v, ones_v, acc):
        cid = lax.axis_index("c")
        sid = lax.axis_index("s")
        w = cid * NS + sid
        sl = pl.ds(sid * ROWS_PT, ROWS_PT)
        pltpu.sync_copy(zeros_hbm, acc.at[sl])
        pltpu.sync_copy(row_hbm.at[w], row_v)
        pltpu.sync_copy(ones_hbm, ones_v)
        plsc.subcore_barrier()

        def body(g, carry):
            pltpu.sync_copy(ones_v, acc.at[row_v.at[g]], add=True)
            return carry

        lax.fori_loop(0, NGRP, body, 0)
        plsc.subcore_barrier()
        pltpu.sync_copy(acc.at[sl], out_hbm.at[cid, sl])

    return deg


_prop64 = _make_prop(64, staged=False)
_prop32 = _make_prop(32, staged=True)
_deg_kernel = _make_deg()


def _dot(a, b):
    return jax.lax.dot_general(a, b, (((1,), (0,)), ((), ())),
                               preferred_element_type=jnp.float32)


def _pad_rows(u):
    return jnp.concatenate(
        [u, jnp.zeros((N_PAD - N, u.shape[1]), jnp.float32)], axis=0)


def _pad_u(u, fpad):
    fo = u.shape[1]
    if fpad > fo:
        u = jnp.concatenate([u, jnp.zeros((N, fpad - fo), jnp.float32)],
                            axis=1)
    return _pad_rows(u)


def _stage_a_body(degp_ref, x_ref, w_ref, dis_ref, u0a_ref, u0b_ref,
                  acc0_ref):
    d = degp_ref[0] + degp_ref[1]
    deg = d[:N, 0:1]
    dis = jnp.where(deg > 0, 1.0 / jnp.sqrt(jnp.maximum(deg, 1e-12)), 0.0)
    x = x_ref[...]
    dis_ref[...] = dis
    u0 = x * dis
    u0a_ref[...] = _pad_rows(u0[:, :64])
    u0b_ref[...] = _pad_rows(u0[:, 64:])
    acc0_ref[...] = _dot(x, w_ref[...])


def _stage_a(degp, x, w10):
    return pl.pallas_call(
        _stage_a_body,
        out_shape=[
            jax.ShapeDtypeStruct((N, 1), jnp.float32),
            jax.ShapeDtypeStruct((N_PAD, 64), jnp.float32),
            jax.ShapeDtypeStruct((N_PAD, 64), jnp.float32),
            jax.ShapeDtypeStruct((N, w10.shape[1]), jnp.float32),
        ],
    )(degp, x, w10)


def _p1_l1_body(slo_ref, shi_ref, dis_ref, accp_ref, w_ref,
                acc_ref, ua_ref, ub_ref):
    s = jnp.concatenate([(slo_ref[0] + slo_ref[1])[:N],
                         (shi_ref[0] + shi_ref[1])[:N]], axis=1)
    dis = dis_ref[...]
    t = -dis * s
    acc_ref[...] = accp_ref[...] + _dot(t, w_ref[...])
    u = dis * t
    ua_ref[...] = _pad_rows(u[:, :64])
    ub_ref[...] = _pad_rows(u[:, 64:])


def _stage_p1_l1(slo, shi, dis, accp, w):
    return pl.pallas_call(
        _p1_l1_body,
        out_shape=[
            jax.ShapeDtypeStruct((N, w.shape[1]), jnp.float32),
            jax.ShapeDtypeStruct((N_PAD, 64), jnp.float32),
            jax.ShapeDtypeStruct((N_PAD, 64), jnp.float32),
        ],
    )(slo, shi, dis, accp, w)


def _p2_l1_body(slo_ref, shi_ref, dis_ref, accp_ref, w_ref, b_ref,
                tx0_ref, wn_ref, h_ref, u_ref, accn_ref):
    s = jnp.concatenate([(slo_ref[0] + slo_ref[1])[:N],
                         (shi_ref[0] + shi_ref[1])[:N]], axis=1)
    dis = dis_ref[...]
    p = -dis * s
    tx2 = 2.0 * p - tx0_ref[...]
    h = jnp.tanh(accp_ref[...] + _dot(tx2, w_ref[...]) + b_ref[...])
    h_ref[...] = h
    u_ref[...] = _pad_u(dis * h, 64)
    accn_ref[...] = _dot(h, wn_ref[...])


def _stage_p2_l1(slo, shi, dis, accp, w, b, tx0, wn):
    return pl.pallas_call(
        _p2_l1_body,
        out_shape=[
            jax.ShapeDtypeStruct((N, w.shape[1]), jnp.float32),
            jax.ShapeDtypeStruct((N_PAD, 64), jnp.float32),
            jax.ShapeDtypeStruct((N, wn.shape[1]), jnp.float32),
        ],
    )(slo, shi, dis, accp, w, b, tx0, wn)


def _p1_body(fin, fpad, sp_ref, dis_ref, accp_ref, w_ref, acc_ref, u_ref):
    s = (sp_ref[0] + sp_ref[1])[:N, :fin]
    dis = dis_ref[...]
    t = -dis * s
    acc_ref[...] = accp_ref[...] + _dot(t, w_ref[...])
    u_ref[...] = _pad_u(dis * t, fpad)


def _stage_p1(sp, dis, accp, w, fin, fpad):
    return pl.pallas_call(
        functools.partial(_p1_body, fin, fpad),
        out_shape=[
            jax.ShapeDtypeStruct((N, w.shape[1]), jnp.float32),
            jax.ShapeDtypeStruct((N_PAD, fpad), jnp.float32),
        ],
    )(sp, dis, accp, w)


def _p2_body(fin, fpad_next, sp_ref, dis_ref, accp_ref, w_ref, b_ref,
             tx0_ref, wn_ref, h_ref, u_ref, accn_ref):
    s = (sp_ref[0] + sp_ref[1])[:N, :fin]
    dis = dis_ref[...]
    p = -dis * s
    tx2 = 2.0 * p - tx0_ref[...]
    h = jnp.tanh(accp_ref[...] + _dot(tx2, w_ref[...]) + b_ref[...])
    h_ref[...] = h
    u_ref[...] = _pad_u(dis * h, fpad_next)
    accn_ref[...] = _dot(h, wn_ref[...])


def _stage_p2(sp, dis, accp, w, b, tx0, wn, fin, fpad_next):
    fo = w.shape[1]
    return pl.pallas_call(
        functools.partial(_p2_body, fin, fpad_next),
        out_shape=[
            jax.ShapeDtypeStruct((N, fo), jnp.float32),
            jax.ShapeDtypeStruct((N_PAD, fpad_next), jnp.float32),
            jax.ShapeDtypeStruct((N, wn.shape[1]), jnp.float32),
        ],
    )(sp, dis, accp, w, b, tx0, wn)


def _p2_last_body(fin, sp_ref, dis_ref, accp_ref, w_ref, b_ref, tx0_ref,
                  h_ref):
    s = (sp_ref[0] + sp_ref[1])[:N, :fin]
    dis = dis_ref[...]
    p = -dis * s
    tx2 = 2.0 * p - tx0_ref[...]
    h_ref[...] = jnp.tanh(accp_ref[...] + _dot(tx2, w_ref[...]) + b_ref[...])


def _stage_p2_last(sp, dis, accp, w, b, tx0, fin):
    return pl.pallas_call(
        functools.partial(_p2_last_body, fin),
        out_shape=jax.ShapeDtypeStruct((N, w.shape[1]), jnp.float32),
    )(sp, dis, accp, w, b, tx0)


def _head_body(concat_ref, aW_ref, ab_ref, cW_ref, cb_ref, logits_ref,
               values_ref):
    a = pl.program_id(0)
    k = pl.program_id(1)
    c = concat_ref[...]
    w = aW_ref[...]
    rows_valid = (jax.lax.broadcasted_iota(jnp.int32, (_KB, 1), 0)
                  + k * _KB) < _KTOT
    w = jnp.where(rows_valid, w, 0.0)
    part = _dot(c, w)

    @pl.when(k == 0)
    def _():
        logits_ref[...] = ab_ref[...]

    logits_ref[...] += part

    @pl.when(a == 0)
    def _():
        cw = jnp.where(rows_valid, cW_ref[...], 0.0)
        cpart = jax.lax.dot_general(c, cw, (((1,), (0,)), ((), ())),
                                    preferred_element_type=jnp.float32,
                                    precision=jax.lax.Precision.HIGHEST)

        @pl.when(k == 0)
        def _():
            values_ref[...] = cb_ref[...]

        values_ref[...] += cpart


def _head(concat_pad, actor_W, actor_b, critic_W, critic_b):
    return pl.pallas_call(
        _head_body,
        grid=(_A_BLOCKS, _K_BLOCKS),
        in_specs=[
            pl.BlockSpec((1, _KB), lambda a, k: (0, k)),
            pl.BlockSpec((_KB, _AB), lambda a, k: (k, a)),
            pl.BlockSpec((1, _AB), lambda a, k: (0, a)),
            pl.BlockSpec((_KB, 1), lambda a, k: (k, 0)),
            pl.BlockSpec((1, 1), lambda a, k: (0, 0)),
        ],
        out_specs=[
            pl.BlockSpec((1, _AB), lambda a, k: (0, a)),
            pl.BlockSpec((1, 1), lambda a, k: (0, 0)),
        ],
        out_shape=[
            jax.ShapeDtypeStruct((1, A), jnp.float32),
            jax.ShapeDtypeStruct((1, 1), jnp.float32),
        ],
    )(concat_pad, actor_W, actor_b, critic_W, critic_b)


def kernel(substrate_features, edge_index, v_CPU_request, v_BW_demand,
           pending_v_nodes, W1, b1, W2, b2, W3, b3,
           actor_W, actor_b, critic_W, critic_b):
    x = substrate_features[0]
    row = edge_index[0]
    col = edge_index[1]

    pad = E_PAD - E
    col_p = jnp.concatenate([col, jnp.zeros((pad,), jnp.int32)])
    col_p = col_p.reshape(NW, NGRP, EPG)
    row_p = jnp.concatenate([row, jnp.full((pad,), N, jnp.int32)])
    row_p = row_p.reshape(NW, NGRP, EPG)

    z8 = jnp.zeros((ROWS_PT, _DEG_F), jnp.float32)
    z64 = jnp.zeros((ROWS_PT, 64), jnp.float32)
    z32 = jnp.zeros((ROWS_PT, 32), jnp.float32)
    ones8 = jnp.concatenate(
        [jnp.ones((EPG, 1), jnp.float32),
         jnp.zeros((EPG, _DEG_F - 1), jnp.float32)], axis=1)

    degp = _deg_kernel(row_p, ones8, z8)

    dis, u0a, u0b, acc0 = _stage_a(degp, x, W1[0])
    s1a = _prop64(col_p, row_p, u0a, z64)
    s1b = _prop64(col_p, row_p, u0b, z64)
    acc1, u1a, u1b = _stage_p1_l1(s1a, s1b, dis, acc0, W1[1])
    s2a = _prop64(col_p, row_p, u1a, z64)
    s2b = _prop64(col_p, row_p, u1b, z64)
    h1, u2, acc2 = _stage_p2_l1(s2a, s2b, dis, acc1, W1[2], b1[None, :], x,
                                W2[0])

    s3 = _prop64(col_p, row_p, u2, z64)
    acc3, u3 = _stage_p1(s3, dis, acc2, W2[1], 60, 64)
    s4 = _prop64(col_p, row_p, u3, z64)
    h2, u4, acc4 = _stage_p2(s4, dis, acc3, W2[2], b2[None, :], h1, W3[0],
                             60, 32)

    s5 = _prop32(col_p, row_p, u4, z32)
    acc5, u5 = _stage_p1(s5, dis, acc4, W3[1], 30, 32)
    s6 = _prop32(col_p, row_p, u5, z32)
    h3 = _stage_p2_last(s6, dis, acc5, W3[2], b3[None, :], h2, 30)

    concat = jnp.concatenate([
        h3[:, 0], v_CPU_request, v_BW_demand, pending_v_nodes,
        jnp.zeros((_K_BLOCKS * _KB - _KTOT,), jnp.float32),
    ])[None, :]
    logits, values = _head(concat, actor_W, actor_b[None, :],
                           critic_W, critic_b[None, :])
    return logits, values
